# baseline (speedup 1.0000x reference)
import math

import jax
import jax.numpy as jnp
from jax.experimental import pallas as pl
from jax.experimental.pallas import tpu as pltpu

_BLUR = 0.05
_EPS = _BLUR ** 2
_INV_EPS = 1.0 / _EPS
_N_ITERS = 10
_LN2 = math.log(2.0)
_L2E = 1.0 / _LN2


def kernel(y_pred, y_true, length_pred, length_true):
    B, T = y_pred.shape
    Tm2 = T - 2
    BLK = 128
    G = B // BLK
    log_a = math.log(1.0 / T)

    def _body(ypt_ref, ytt_ref, lp_ref, lt_ref, out_ref, t_ref):
        yp = ypt_ref[1:T - 1, :]
        yt = ytt_ref[1:T - 1, :]
        c2 = 0.5 * _INV_EPS * _L2E
        sc = math.sqrt(c2)
        Xs = jnp.stack([yt, yt, yp]) * sc
        Ys = jnp.stack([yp, yt, yp]) * sc
        Ce = (Xs[:, :, None, :] - Ys[:, None, :, :]) ** 2
        CeT = (Ys[:, :, None, :] - Xs[:, None, :, :]) ** 2

        la2 = log_a * _L2E

        def lse1(F, Cm):
            t0 = F[:, 0, None, :] - Cm[:, 0]
            t_ref[:, 0] = t0
            m = t0
            for i in range(1, Tm2):
                ti = F[:, i, None, :] - Cm[:, i]
                t_ref[:, i] = ti
                m = jnp.maximum(m, ti)
            s = jnp.exp2(t_ref[:, 0] - m)
            for i in range(1, Tm2):
                s = s + jnp.exp2(t_ref[:, i] - m)
            return m + jnp.log2(s)

        f = jnp.zeros((3, Tm2, BLK), jnp.float32)
        g = f
        for _ in range(_N_ITERS):
            g = -(lse1(f, Ce) + la2)
            f = -(lse1(g, CeT) + la2)
        ot = (jnp.sum(f, axis=1) + jnp.sum(g, axis=1)) * (_EPS * _LN2 / T)
        div = ot[0] - 0.5 * ot[1] - 0.5 * ot[2]
        tim = jnp.sum((yp - yt) ** 2, axis=0)
        dl = lp_ref[0, :] - lt_ref[0, :]
        out_ref[0] = jnp.stack([div, tim, dl * dl])

    out = pl.pallas_call(
        _body,
        grid=(G,),
        in_specs=[
            pl.BlockSpec((T, BLK), lambda i: (0, i)),
            pl.BlockSpec((T, BLK), lambda i: (0, i)),
            pl.BlockSpec((1, BLK), lambda i: (0, i)),
            pl.BlockSpec((1, BLK), lambda i: (0, i)),
        ],
        out_specs=pl.BlockSpec((1, 3, BLK), lambda i: (i, 0, 0)),
        out_shape=jax.ShapeDtypeStruct((G, 3, BLK), jnp.float32),
        scratch_shapes=[pltpu.VMEM((3, Tm2, Tm2, BLK), jnp.float32)],
        compiler_params=pltpu.CompilerParams(
            dimension_semantics=("parallel",)),
    )(y_pred.T, y_true.T, length_pred.reshape(1, B), length_true.reshape(1, B))

    sums = out.sum(axis=(0, 2))
    distrib_loss = sums[0] / B
    timing_loss = sums[1] / (B * Tm2)
    length_loss = sums[2] / B
    weighted_loss = timing_loss + length_loss + distrib_loss
    return (weighted_loss, length_loss, timing_loss)


# speedup vs baseline: 1.0595x; 1.0595x over previous
import math

import jax
import jax.numpy as jnp
from jax.experimental import pallas as pl
from jax.experimental.pallas import tpu as pltpu

_BLUR = 0.05
_EPS = _BLUR ** 2
_INV_EPS = 1.0 / _EPS
_N_ITERS = 10
_LN2 = math.log(2.0)
_L2E = 1.0 / _LN2


def kernel(y_pred, y_true, length_pred, length_true):
    B, T = y_pred.shape
    Tm2 = T - 2
    BLK = 128
    G = B // BLK
    log_a = math.log(1.0 / T)

    def _body(ypt_ref, ytt_ref, lp_ref, lt_ref, out_ref):
        yp = ypt_ref[1:T - 1, :]
        yt = ytt_ref[1:T - 1, :]
        c2 = 0.5 * _INV_EPS * _L2E
        sc = math.sqrt(c2)
        Xs = jnp.stack([yt, yt, yp]) * sc
        Ys = jnp.stack([yp, yt, yp]) * sc
        Ce = (Xs[:, :, None, :] - Ys[:, None, :, :]) ** 2
        CeT = (Ys[:, :, None, :] - Xs[:, None, :, :]) ** 2

        la2 = log_a * _L2E

        def lse1(F, Cm):
            m = F[:, 0, None, :] - Cm[:, 0]
            for i in range(1, Tm2):
                m = jnp.maximum(m, F[:, i, None, :] - Cm[:, i])
            s = jnp.exp2(F[:, 0, None, :] - (Cm[:, 0] + m))
            for i in range(1, Tm2):
                s = s + jnp.exp2(F[:, i, None, :] - (Cm[:, i] + m))
            return m + jnp.log2(s)

        f = jnp.zeros((3, Tm2, BLK), jnp.float32)
        g = f
        for _ in range(_N_ITERS):
            g = -(lse1(f, Ce) + la2)
            f = -(lse1(g, CeT) + la2)
        ot = (jnp.sum(f, axis=1) + jnp.sum(g, axis=1)) * (_EPS * _LN2 / T)
        div = ot[0] - 0.5 * ot[1] - 0.5 * ot[2]
        tim = jnp.sum((yp - yt) ** 2, axis=0)
        dl = lp_ref[0, :] - lt_ref[0, :]
        out_ref[0] = jnp.stack([div, tim, dl * dl])

    out = pl.pallas_call(
        _body,
        grid=(G,),
        in_specs=[
            pl.BlockSpec((T, BLK), lambda i: (0, i)),
            pl.BlockSpec((T, BLK), lambda i: (0, i)),
            pl.BlockSpec((1, BLK), lambda i: (0, i)),
            pl.BlockSpec((1, BLK), lambda i: (0, i)),
        ],
        out_specs=pl.BlockSpec((1, 3, BLK), lambda i: (i, 0, 0)),
        out_shape=jax.ShapeDtypeStruct((G, 3, BLK), jnp.float32),
        compiler_params=pltpu.CompilerParams(
            dimension_semantics=("parallel",)),
    )(y_pred.T, y_true.T, length_pred.reshape(1, B), length_true.reshape(1, B))

    sums = out.sum(axis=(0, 2))
    distrib_loss = sums[0] / B
    timing_loss = sums[1] / (B * Tm2)
    length_loss = sums[2] / B
    weighted_loss = timing_loss + length_loss + distrib_loss
    return (weighted_loss, length_loss, timing_loss)


# speedup vs baseline: 1.0813x; 1.0206x over previous
import math

import jax
import jax.numpy as jnp
from jax.experimental import pallas as pl
from jax.experimental.pallas import tpu as pltpu

_BLUR = 0.05
_EPS = _BLUR ** 2
_INV_EPS = 1.0 / _EPS
_N_ITERS = 10
_LN2 = math.log(2.0)
_L2E = 1.0 / _LN2


def kernel(y_pred, y_true, length_pred, length_true):
    B, T = y_pred.shape
    Tm2 = T - 2
    BLK = 128
    G = B // BLK
    log_a = math.log(1.0 / T)

    def _body(ypt_ref, ytt_ref, lp_ref, lt_ref, out_ref):
        yp = ypt_ref[1:T - 1, :]
        yt = ytt_ref[1:T - 1, :]
        c2 = 0.5 * _INV_EPS * _L2E
        sc = math.sqrt(c2)
        xs = yt * sc
        ys = yp * sc
        Ce0 = (xs[:, None, :] - ys[None, :, :]) ** 2
        Ce0T = (ys[:, None, :] - xs[None, :, :]) ** 2
        Ce1 = (xs[:, None, :] - xs[None, :, :]) ** 2
        Ce2 = (ys[:, None, :] - ys[None, :, :]) ** 2

        la2 = log_a * _L2E

        def lse1(F, Cm):
            m = F[0, None, :] - Cm[0]
            for i in range(1, Tm2):
                m = jnp.maximum(m, F[i, None, :] - Cm[i])
            s = jnp.exp2(F[0, None, :] - (Cm[0] + m))
            for i in range(1, Tm2):
                s = s + jnp.exp2(F[i, None, :] - (Cm[i] + m))
            return m + jnp.log2(s)

        z = jnp.zeros((Tm2, BLK), jnp.float32)
        f0, f1, f2 = z, z, z
        for _ in range(_N_ITERS):
            g0 = -(lse1(f0, Ce0) + la2)
            g1 = -(lse1(f1, Ce1) + la2)
            g2 = -(lse1(f2, Ce2) + la2)
            f0 = -(lse1(g0, Ce0T) + la2)
            f1 = -(lse1(g1, Ce1) + la2)
            f2 = -(lse1(g2, Ce2) + la2)
        w = _EPS * _LN2 / T
        ot0 = (jnp.sum(f0, axis=0) + jnp.sum(g0, axis=0)) * w
        ot1 = (jnp.sum(f1, axis=0) + jnp.sum(g1, axis=0)) * w
        ot2 = (jnp.sum(f2, axis=0) + jnp.sum(g2, axis=0)) * w
        div = ot0 - 0.5 * ot1 - 0.5 * ot2
        tim = jnp.sum((yp - yt) ** 2, axis=0)
        dl = lp_ref[0, :] - lt_ref[0, :]
        out_ref[0] = jnp.stack([div, tim, dl * dl])

    out = pl.pallas_call(
        _body,
        grid=(G,),
        in_specs=[
            pl.BlockSpec((T, BLK), lambda i: (0, i)),
            pl.BlockSpec((T, BLK), lambda i: (0, i)),
            pl.BlockSpec((1, BLK), lambda i: (0, i)),
            pl.BlockSpec((1, BLK), lambda i: (0, i)),
        ],
        out_specs=pl.BlockSpec((1, 3, BLK), lambda i: (i, 0, 0)),
        out_shape=jax.ShapeDtypeStruct((G, 3, BLK), jnp.float32),
        compiler_params=pltpu.CompilerParams(
            dimension_semantics=("parallel",)),
    )(y_pred.T, y_true.T, length_pred.reshape(1, B), length_true.reshape(1, B))

    sums = out.sum(axis=(0, 2))
    distrib_loss = sums[0] / B
    timing_loss = sums[1] / (B * Tm2)
    length_loss = sums[2] / B
    weighted_loss = timing_loss + length_loss + distrib_loss
    return (weighted_loss, length_loss, timing_loss)


# speedup vs baseline: 1.1395x; 1.0538x over previous
import math

import jax
import jax.numpy as jnp
from jax.experimental import pallas as pl
from jax.experimental.pallas import tpu as pltpu

_BLUR = 0.05
_EPS = _BLUR ** 2
_INV_EPS = 1.0 / _EPS
_N_ITERS = 10
_LN2 = math.log(2.0)
_L2E = 1.0 / _LN2


def kernel(y_pred, y_true, length_pred, length_true):
    B, T = y_pred.shape
    Tm2 = T - 2
    BLK = 128
    G = B // BLK
    log_a = math.log(1.0 / T)

    def _body(ypt_ref, ytt_ref, lp_ref, lt_ref, out_ref, t0_ref, t1_ref,
              t2_ref):
        yp = ypt_ref[1:T - 1, :]
        yt = ytt_ref[1:T - 1, :]
        c2 = 0.5 * _INV_EPS * _L2E
        sc = math.sqrt(c2)
        xs = yt * sc
        ys = yp * sc
        Ce0 = (xs[:, None, :] - ys[None, :, :]) ** 2
        Ce0T = (ys[:, None, :] - xs[None, :, :]) ** 2
        Ce1 = (xs[:, None, :] - xs[None, :, :]) ** 2
        Ce2 = (ys[:, None, :] - ys[None, :, :]) ** 2

        la2 = log_a * _L2E

        def lse1(F, Cm, t_ref):
            t = F[0, None, :] - Cm[0]
            t_ref[0] = t
            m = t
            for i in range(1, Tm2):
                t = F[i, None, :] - Cm[i]
                t_ref[i] = t
                m = jnp.maximum(m, t)
            s = jnp.exp2(t_ref[0] - m)
            for i in range(1, Tm2):
                s = s + jnp.exp2(t_ref[i] - m)
            return m + jnp.log2(s)

        z = jnp.zeros((Tm2, BLK), jnp.float32)
        f0, f1, f2 = z, z, z
        for _ in range(_N_ITERS):
            g0 = -(lse1(f0, Ce0, t0_ref) + la2)
            g1 = -(lse1(f1, Ce1, t1_ref) + la2)
            g2 = -(lse1(f2, Ce2, t2_ref) + la2)
            f0 = -(lse1(g0, Ce0T, t0_ref) + la2)
            f1 = -(lse1(g1, Ce1, t1_ref) + la2)
            f2 = -(lse1(g2, Ce2, t2_ref) + la2)
        w = _EPS * _LN2 / T
        ot0 = (jnp.sum(f0, axis=0) + jnp.sum(g0, axis=0)) * w
        ot1 = (jnp.sum(f1, axis=0) + jnp.sum(g1, axis=0)) * w
        ot2 = (jnp.sum(f2, axis=0) + jnp.sum(g2, axis=0)) * w
        div = ot0 - 0.5 * ot1 - 0.5 * ot2
        tim = jnp.sum((yp - yt) ** 2, axis=0)
        dl = lp_ref[0, :] - lt_ref[0, :]
        out_ref[0] = jnp.stack([div, tim, dl * dl])

    out = pl.pallas_call(
        _body,
        grid=(G,),
        in_specs=[
            pl.BlockSpec((T, BLK), lambda i: (0, i)),
            pl.BlockSpec((T, BLK), lambda i: (0, i)),
            pl.BlockSpec((1, BLK), lambda i: (0, i)),
            pl.BlockSpec((1, BLK), lambda i: (0, i)),
        ],
        out_specs=pl.BlockSpec((1, 3, BLK), lambda i: (i, 0, 0)),
        out_shape=jax.ShapeDtypeStruct((G, 3, BLK), jnp.float32),
        scratch_shapes=[pltpu.VMEM((Tm2, Tm2, BLK), jnp.float32)] * 3,
        compiler_params=pltpu.CompilerParams(
            dimension_semantics=("parallel",)),
    )(y_pred.T, y_true.T, length_pred.reshape(1, B), length_true.reshape(1, B))

    sums = out.sum(axis=(0, 2))
    distrib_loss = sums[0] / B
    timing_loss = sums[1] / (B * Tm2)
    length_loss = sums[2] / B
    weighted_loss = timing_loss + length_loss + distrib_loss
    return (weighted_loss, length_loss, timing_loss)
